# Initial kernel scaffold; baseline (speedup 1.0000x reference)
import jax
import jax.numpy as jnp
from jax.experimental import pallas as pl


def kernel(x, emb_table, W, b):
    raise NotImplementedError("write your pallas kernel here")


# speedup vs baseline: 5.0833x; 5.0833x over previous
import functools

import jax
import jax.numpy as jnp
from jax import lax
from jax.experimental import pallas as pl
from jax.experimental.pallas import tpu as pltpu
from jax.experimental.pallas import tpu_sc as plsc

_CHUNK = 128


@functools.partial(jax.jit, static_argnames=("num_workers",))
def _sc_gather(emb_table, idx2d, num_workers=32):
    _, c_per_w, chunk = idx2d.shape
    rows = num_workers * c_per_w * chunk
    _, d = emb_table.shape
    r_per_w = c_per_w * chunk
    mesh = plsc.VectorSubcoreMesh(core_axis_name="c", subcore_axis_name="s")
    nc = 2

    @functools.partial(
        pl.kernel,
        mesh=mesh,
        out_type=jax.ShapeDtypeStruct((rows, d), jnp.float32),
        scratch_types=[
            pltpu.VMEM((c_per_w, chunk), jnp.int32),
            pltpu.VMEM((r_per_w, d), jnp.float32),
            pltpu.SemaphoreType.DMA,
        ],
        compiler_params=pltpu.CompilerParams(use_tc_tiling_on_sc=False),
    )
    def gather_kernel(table_hbm, idx_hbm, out_hbm, idx_v, rows_v, sem):
        wid = lax.axis_index("s") * nc + lax.axis_index("c")
        rbase = wid * r_per_w
        pltpu.sync_copy(idx_hbm.at[wid], idx_v)
        copies = []
        for j in range(c_per_w):
            copies.append(
                pltpu.async_copy(
                    table_hbm.at[idx_v.at[j]],
                    rows_v.at[pl.ds(j * chunk, chunk)],
                    sem,
                )
            )
        for c in copies:
            c.wait()
        pltpu.sync_copy(rows_v, out_hbm.at[pl.ds(rbase, r_per_w)])

    return gather_kernel(emb_table, idx2d)


def _linear_body(f_ref, w_ref, b_ref, o_ref):
    f = jnp.maximum(f_ref[...], 0.0)
    o_ref[...] = (
        lax.dot_general(
            f,
            w_ref[...],
            (((1,), (1,)), ((), ())),
            preferred_element_type=jnp.float32,
        )
        + b_ref[...]
    )


@jax.jit
def _tc_linear(feats, w, b):
    batch, fan_in = feats.shape
    t = w.shape[0]
    bt = 512
    grid = (batch // bt,)
    return pl.pallas_call(
        _linear_body,
        grid=grid,
        in_specs=[
            pl.BlockSpec((bt, fan_in), lambda i: (i, 0)),
            pl.BlockSpec((t, fan_in), lambda i: (0, 0)),
            pl.BlockSpec((1, t), lambda i: (0, 0)),
        ],
        out_specs=pl.BlockSpec((bt, t), lambda i: (i, 0)),
        out_shape=jax.ShapeDtypeStruct((batch, t), jnp.float32),
    )(feats, w, b.reshape(1, t))


def kernel(x, emb_table, W, b):
    batch, inp = x.shape
    _, d = emb_table.shape
    rows = batch * inp
    nw = 32
    idx2d = x.reshape(nw, rows // (_CHUNK * nw), _CHUNK)
    feats = _sc_gather(emb_table, idx2d)
    f = feats.reshape(batch, inp * d)
    return _tc_linear(f, W, b)


# speedup vs baseline: 5.1463x; 1.0124x over previous
import functools

import jax
import jax.numpy as jnp
from jax import lax
from jax.experimental import pallas as pl
from jax.experimental.pallas import tpu as pltpu
from jax.experimental.pallas import tpu_sc as plsc

_CHUNK = 128
_NW = 32
_SLOTS = 8


@jax.jit
def _sc_gather(emb_table, idx4d):
    _, n_pass, c_per_pass, chunk = idx4d.shape
    rows = _NW * n_pass * c_per_pass * chunk
    _, d = emb_table.shape
    r_per_pass = c_per_pass * chunk
    b_per_pass = r_per_pass * d // (_SLOTS * 128)
    batch = rows * d // (_SLOTS * 128)
    mesh = plsc.VectorSubcoreMesh(core_axis_name="c", subcore_axis_name="s")
    nc = 2

    @functools.partial(
        pl.kernel,
        mesh=mesh,
        out_type=jax.ShapeDtypeStruct((rows, d), jnp.float32),
        scratch_types=[
            pltpu.VMEM((n_pass, c_per_pass, chunk), jnp.int32),
            pltpu.VMEM((r_per_pass, d), jnp.float32),
            pltpu.SemaphoreType.DMA,
        ],
        compiler_params=pltpu.CompilerParams(use_tc_tiling_on_sc=False),
    )
    def gather_kernel(table_hbm, idx_hbm, out_hbm, idx_v, rows_v, sem):
        wid = lax.axis_index("s") * nc + lax.axis_index("c")
        pltpu.sync_copy(idx_hbm.at[wid], idx_v)
        for p in range(n_pass):
            copies = []
            for j in range(c_per_pass):
                copies.append(
                    pltpu.async_copy(
                        table_hbm.at[idx_v.at[p, j]],
                        rows_v.at[pl.ds(j * chunk, chunk)],
                        sem,
                    )
                )
            for c in copies:
                c.wait()
            pltpu.sync_copy(
                rows_v,
                out_hbm.at[pl.ds((wid * n_pass + p) * r_per_pass, r_per_pass)],
            )

    return gather_kernel(emb_table, idx4d).reshape(batch, _SLOTS, 128)


def _linear_body(f_ref, w_ref, b_ref, o_ref):
    acc = b_ref[...]
    for s in range(_SLOTS):
        f = jnp.maximum(f_ref[:, s, :], 0.0)
        acc = acc + jnp.dot(f, w_ref[s], preferred_element_type=jnp.float32)
    o_ref[...] = acc


@jax.jit
def _tc_linear(feats, wr, b):
    batch = feats.shape[0]
    t = wr.shape[2]
    bt = 512
    grid = (batch // bt,)
    return pl.pallas_call(
        _linear_body,
        grid=grid,
        in_specs=[
            pl.BlockSpec((bt, _SLOTS, 128), lambda i: (i, 0, 0)),
            pl.BlockSpec((_SLOTS, 128, t), lambda i: (0, 0, 0)),
            pl.BlockSpec((1, t), lambda i: (0, 0)),
        ],
        out_specs=pl.BlockSpec((bt, t), lambda i: (i, 0)),
        out_shape=jax.ShapeDtypeStruct((batch, t), jnp.float32),
    )(feats, wr, b.reshape(1, t))


def kernel(x, emb_table, W, b):
    batch, inp = x.shape
    _, d = emb_table.shape
    t = W.shape[0]
    slots = _SLOTS * 128 // d
    xp = jnp.concatenate(
        [x, jnp.broadcast_to(x[:, :1], (batch, slots - inp))], axis=1
    )
    rows = batch * slots
    n_pass = 2
    c_per_pass = rows // (_NW * n_pass * _CHUNK)
    idx4d = xp.reshape(_NW, n_pass, c_per_pass, _CHUNK)
    feats = _sc_gather(emb_table, idx4d)
    wr = jnp.pad(W.T, ((0, (slots - inp) * d), (0, 0))).reshape(_SLOTS, 128, t)
    return _tc_linear(feats, wr, b)


# speedup vs baseline: 9.4797x; 1.8420x over previous
import functools

import jax
import jax.numpy as jnp
from jax import lax
from jax.experimental import pallas as pl
from jax.experimental.pallas import tpu as pltpu
from jax.experimental.pallas import tpu_sc as plsc

_CHUNK = 128
_NW = 32
_SLOTS = 8


@jax.jit
def _sc_gather(emb_table, idx4d):
    _, n_pass, c_per_pass, chunk = idx4d.shape
    rows = _NW * n_pass * c_per_pass * chunk
    _, d = emb_table.shape
    r_per_pass = c_per_pass * chunk
    b_per_pass = r_per_pass * d // (_SLOTS * 128)
    batch = rows * d // (_SLOTS * 128)
    mesh = plsc.VectorSubcoreMesh(core_axis_name="c", subcore_axis_name="s")
    nc = 2

    @functools.partial(
        pl.kernel,
        mesh=mesh,
        out_type=jax.ShapeDtypeStruct((rows, d), jnp.float32),
        scratch_types=[
            pltpu.VMEM((n_pass, c_per_pass, chunk), jnp.int32),
            pltpu.VMEM((r_per_pass, d), jnp.float32),
            pltpu.SemaphoreType.DMA,
        ],
        compiler_params=pltpu.CompilerParams(use_tc_tiling_on_sc=False),
    )
    def gather_kernel(table_hbm, idx_hbm, out_hbm, idx_v, rows_v, sem):
        wid = lax.axis_index("s") * nc + lax.axis_index("c")
        pltpu.sync_copy(idx_hbm.at[wid], idx_v)
        for p in range(n_pass):
            copies = []
            for j in range(c_per_pass):
                copies.append(
                    pltpu.async_copy(
                        table_hbm.at[idx_v.at[p, j]],
                        rows_v.at[pl.ds(j * chunk, chunk)],
                        sem,
                    )
                )
            for c in copies:
                c.wait()
            pltpu.sync_copy(
                rows_v,
                out_hbm.at[pl.ds((wid * n_pass + p) * r_per_pass, r_per_pass)],
            )

    return gather_kernel(emb_table, idx4d).reshape(batch, _SLOTS, 128)


_R = 4096
_RB = 12


def _repack_body(t_ref, o_ref):
    r = t_ref.shape[1] // 4
    for g in range(4):
        o_ref[:, g * 32 : (g + 1) * 32] = jnp.swapaxes(
            t_ref[:, g * r : (g + 1) * r], 0, 1
        )


@jax.jit
def _tc_repack(tT):
    d, vocab = tT.shape
    n_j = -(-vocab // _R)
    n_i = -(-n_j // 4)
    return pl.pallas_call(
        _repack_body,
        grid=(n_i,),
        in_specs=[pl.BlockSpec((d, 4 * _R), lambda i: (0, i))],
        out_specs=pl.BlockSpec((_R, 128), lambda i: (i, 0)),
        out_shape=jax.ShapeDtypeStruct((n_i * _R, 128), jnp.float32),
    )(tT)


def _linear_body(f_ref, w_ref, b_ref, o_ref):
    acc = b_ref[...]
    for s in range(_SLOTS):
        f = jnp.maximum(f_ref[:, s, :], 0.0)
        acc = acc + jnp.dot(f, w_ref[s], preferred_element_type=jnp.float32)
    o_ref[...] = acc


@jax.jit
def _tc_linear(feats, wr, b):
    batch = feats.shape[0]
    t = wr.shape[2]
    bt = 512
    grid = (batch // bt,)
    return pl.pallas_call(
        _linear_body,
        grid=grid,
        in_specs=[
            pl.BlockSpec((bt, _SLOTS, 128), lambda i: (i, 0, 0)),
            pl.BlockSpec((_SLOTS, 128, t), lambda i: (0, 0, 0)),
            pl.BlockSpec((1, t), lambda i: (0, 0)),
        ],
        out_specs=pl.BlockSpec((bt, t), lambda i: (i, 0)),
        out_shape=jax.ShapeDtypeStruct((batch, t), jnp.float32),
    )(feats, wr, b.reshape(1, t))


def kernel(x, emb_table, W, b):
    batch, inp = x.shape
    _, d = emb_table.shape
    t = W.shape[0]
    slots = _SLOTS * 128 // d
    xp = jnp.concatenate(
        [x, jnp.broadcast_to(x[:, :1], (batch, slots - inp))], axis=1
    )
    fx = (
        ((xp >> (_RB + 2)) << (_RB + 2))
        + ((xp & (_R - 1)) << 2)
        + ((xp >> _RB) & 3)
    )
    rows = batch * slots
    n_pass = 2
    c_per_pass = rows // (_NW * n_pass * _CHUNK)
    idx4d = fx.reshape(_NW, n_pass, c_per_pass, _CHUNK)
    t2 = _tc_repack(jnp.swapaxes(emb_table, 0, 1))
    t_lin = jnp.reshape(t2, (t2.shape[0] * 4, d))
    feats = _sc_gather(t_lin, idx4d)
    wr = jnp.pad(W.T, ((0, (slots - inp) * d), (0, 0))).reshape(_SLOTS, 128, t)
    return _tc_linear(feats, wr, b)


# speedup vs baseline: 14.9487x; 1.5769x over previous
import functools

import jax
import jax.numpy as jnp
from jax import lax
from jax.experimental import pallas as pl
from jax.experimental.pallas import tpu as pltpu
from jax.experimental.pallas import tpu_sc as plsc

_CHUNK = 128
_NW = 32
_SLOTS = 8


@jax.jit
def _sc_gather(emb_table, idx4d):
    _, n_pass, c_per_pass, chunk = idx4d.shape
    rows = _NW * n_pass * c_per_pass * chunk
    _, d = emb_table.shape
    r_per_pass = c_per_pass * chunk
    b_per_pass = r_per_pass * d // (_SLOTS * 128)
    batch = rows * d // (_SLOTS * 128)
    mesh = plsc.VectorSubcoreMesh(core_axis_name="c", subcore_axis_name="s")
    nc = 2

    @functools.partial(
        pl.kernel,
        mesh=mesh,
        out_type=jax.ShapeDtypeStruct((rows, d), jnp.float32),
        scratch_types=[
            pltpu.VMEM((n_pass, c_per_pass, chunk), jnp.int32),
            pltpu.VMEM((r_per_pass, d), jnp.float32),
            pltpu.SemaphoreType.DMA,
        ],
        compiler_params=pltpu.CompilerParams(use_tc_tiling_on_sc=False),
    )
    def gather_kernel(table_hbm, idx_hbm, out_hbm, idx_v, rows_v, sem):
        wid = lax.axis_index("s") * nc + lax.axis_index("c")
        pltpu.sync_copy(idx_hbm.at[wid], idx_v)
        for p in range(n_pass):
            copies = []
            for j in range(c_per_pass):
                copies.append(
                    pltpu.async_copy(
                        table_hbm.at[idx_v.at[p, j]],
                        rows_v.at[pl.ds(j * chunk, chunk)],
                        sem,
                    )
                )
            for c in copies:
                c.wait()
            pltpu.sync_copy(
                rows_v,
                out_hbm.at[pl.ds((wid * n_pass + p) * r_per_pass, r_per_pass)],
            )

    return gather_kernel(emb_table, idx4d).reshape(batch, _SLOTS, 128)


_C = 8192


def _repack_body(t_ref, o_ref):
    c = t_ref.shape[1]
    for q in range(c // 512):
        blk = jnp.concatenate(
            [t_ref[:, q * 512 + 128 * k : q * 512 + 128 * (k + 1)] for k in range(4)],
            axis=0,
        )
        o_ref[q * 128 : (q + 1) * 128, :] = blk.T


@jax.jit
def _tc_repack(tT):
    d, vocab = tT.shape
    n_i = -(-vocab // _C)
    return pl.pallas_call(
        _repack_body,
        grid=(n_i,),
        in_specs=[pl.BlockSpec((d, _C), lambda i: (0, i))],
        out_specs=pl.BlockSpec((_C // 4, 128), lambda i: (i, 0)),
        out_shape=jax.ShapeDtypeStruct((n_i * _C // 4, 128), jnp.float32),
    )(tT)


def _linear_body(f_ref, w_ref, b_ref, o_ref):
    acc = b_ref[...]
    for s in range(_SLOTS):
        f = jnp.maximum(f_ref[:, s, :], 0.0)
        acc = acc + jnp.dot(f, w_ref[s], preferred_element_type=jnp.float32)
    o_ref[...] = acc


@jax.jit
def _tc_linear(feats, wr, b):
    batch = feats.shape[0]
    t = wr.shape[2]
    bt = 512
    grid = (batch // bt,)
    return pl.pallas_call(
        _linear_body,
        grid=grid,
        in_specs=[
            pl.BlockSpec((bt, _SLOTS, 128), lambda i: (i, 0, 0)),
            pl.BlockSpec((_SLOTS, 128, t), lambda i: (0, 0, 0)),
            pl.BlockSpec((1, t), lambda i: (0, 0)),
        ],
        out_specs=pl.BlockSpec((bt, t), lambda i: (i, 0)),
        out_shape=jax.ShapeDtypeStruct((batch, t), jnp.float32),
    )(feats, wr, b.reshape(1, t))


def kernel(x, emb_table, W, b):
    batch, inp = x.shape
    _, d = emb_table.shape
    t = W.shape[0]
    slots = _SLOTS * 128 // d
    xp = jnp.concatenate(
        [x, jnp.broadcast_to(x[:, :1], (batch, slots - inp))], axis=1
    )
    fx = ((xp >> 9) << 9) + ((xp & 127) << 2) + ((xp >> 7) & 3)
    rows = batch * slots
    n_pass = 2
    c_per_pass = rows // (_NW * n_pass * _CHUNK)
    idx4d = fx.reshape(_NW, n_pass, c_per_pass, _CHUNK)
    t2 = _tc_repack(jnp.swapaxes(emb_table, 0, 1))
    t_lin = jnp.reshape(t2, (t2.shape[0] * 4, d))
    feats = _sc_gather(t_lin, idx4d)
    wr = jnp.pad(W.T, ((0, (slots - inp) * d), (0, 0))).reshape(_SLOTS, 128, t)
    return _tc_linear(feats, wr, b)


# speedup vs baseline: 18.7689x; 1.2556x over previous
import functools

import jax
import jax.numpy as jnp
from jax import lax
from jax.experimental import pallas as pl
from jax.experimental.pallas import tpu as pltpu
from jax.experimental.pallas import tpu_sc as plsc

_CHUNK = 128
_NW = 32
_SLOTS = 8


@jax.jit
def _sc_gather(emb_table, idx4d):
    _, n_pass, c_per_pass, chunk = idx4d.shape
    rows = _NW * n_pass * c_per_pass * chunk
    _, d = emb_table.shape
    r_per_pass = c_per_pass * chunk
    b_per_pass = r_per_pass * d // (_SLOTS * 128)
    batch = rows * d // (_SLOTS * 128)
    mesh = plsc.VectorSubcoreMesh(core_axis_name="c", subcore_axis_name="s")
    nc = 2

    @functools.partial(
        pl.kernel,
        mesh=mesh,
        out_type=jax.ShapeDtypeStruct((rows, d), jnp.float32),
        scratch_types=[
            pltpu.VMEM((n_pass, c_per_pass, chunk), jnp.int32),
            pltpu.VMEM((r_per_pass, d), jnp.float32),
            pltpu.SemaphoreType.DMA,
        ],
        compiler_params=pltpu.CompilerParams(use_tc_tiling_on_sc=False),
    )
    def gather_kernel(table_hbm, idx_hbm, out_hbm, idx_v, rows_v, sem):
        wid = lax.axis_index("s") * nc + lax.axis_index("c")
        pltpu.sync_copy(idx_hbm.at[wid], idx_v)
        for p in range(n_pass):
            copies = []
            for j in range(c_per_pass):
                copies.append(
                    pltpu.async_copy(
                        table_hbm.at[idx_v.at[p, j]],
                        rows_v.at[pl.ds(j * chunk, chunk)],
                        sem,
                    )
                )
            for c in copies:
                c.wait()
            pltpu.sync_copy(
                rows_v,
                out_hbm.at[pl.ds((wid * n_pass + p) * r_per_pass, r_per_pass)],
            )

    return gather_kernel(emb_table, idx4d).reshape(batch, _SLOTS, 128)


_C = 16384


def _repack_body(t_ref, o_ref):
    c = t_ref.shape[1]
    for q in range(c // 512):
        blk = jnp.concatenate(
            [t_ref[:, q * 512 + 128 * k : q * 512 + 128 * (k + 1)] for k in range(4)],
            axis=0,
        )
        o_ref[q * 128 : (q + 1) * 128, :] = blk.T


@jax.jit
def _tc_repack(tT):
    d, vocab = tT.shape
    n_i = -(-vocab // _C)
    return pl.pallas_call(
        _repack_body,
        grid=(n_i,),
        in_specs=[pl.BlockSpec((d, _C), lambda i: (0, i))],
        out_specs=pl.BlockSpec((_C // 4, 128), lambda i: (i, 0)),
        out_shape=jax.ShapeDtypeStruct((n_i * _C // 4, 128), jnp.float32),
    )(tT)


def _linear_body(f_ref, w_ref, b_ref, o_ref):
    acc = b_ref[...]
    for s in range(_SLOTS):
        f = jnp.maximum(f_ref[:, s, :], 0.0)
        acc = acc + jnp.dot(f, w_ref[s], preferred_element_type=jnp.float32)
    o_ref[...] = acc


@jax.jit
def _tc_linear(feats, wr, b):
    batch = feats.shape[0]
    t = wr.shape[2]
    bt = 512
    grid = (batch // bt,)
    return pl.pallas_call(
        _linear_body,
        grid=grid,
        in_specs=[
            pl.BlockSpec((bt, _SLOTS, 128), lambda i: (i, 0, 0)),
            pl.BlockSpec((_SLOTS, 128, t), lambda i: (0, 0, 0)),
            pl.BlockSpec((1, t), lambda i: (0, 0)),
        ],
        out_specs=pl.BlockSpec((bt, t), lambda i: (i, 0)),
        out_shape=jax.ShapeDtypeStruct((batch, t), jnp.float32),
    )(feats, wr, b.reshape(1, t))


def kernel(x, emb_table, W, b):
    batch, inp = x.shape
    _, d = emb_table.shape
    t = W.shape[0]
    slots = _SLOTS * 128 // d
    xp = jnp.concatenate(
        [x, jnp.broadcast_to(x[:, :1], (batch, slots - inp))], axis=1
    )
    fx = ((xp >> 9) << 9) + ((xp & 127) << 2) + ((xp >> 7) & 3)
    rows = batch * slots
    n_pass = 2
    c_per_pass = rows // (_NW * n_pass * _CHUNK)
    idx4d = fx.reshape(_NW, n_pass, c_per_pass, _CHUNK)
    t2 = _tc_repack(jnp.swapaxes(emb_table, 0, 1))
    t_lin = jnp.reshape(t2, (t2.shape[0] * 4, d))
    feats = _sc_gather(t_lin, idx4d)
    wr = jnp.pad(W.T, ((0, (slots - inp) * d), (0, 0))).reshape(_SLOTS, 128, t)
    return _tc_linear(feats, wr, b)


# speedup vs baseline: 20.8273x; 1.1097x over previous
import functools

import jax
import jax.numpy as jnp
from jax import lax
from jax.experimental import pallas as pl
from jax.experimental.pallas import tpu as pltpu
from jax.experimental.pallas import tpu_sc as plsc

_CHUNK = 128
_NW = 32
_SLOTS = 8


@jax.jit
def _sc_gather(emb_table, idx4d):
    _, n_pass, c_per_pass, chunk = idx4d.shape
    rows = _NW * n_pass * c_per_pass * chunk
    _, d = emb_table.shape
    r_per_pass = c_per_pass * chunk
    b_per_pass = r_per_pass * d // (_SLOTS * 128)
    batch = rows * d // (_SLOTS * 128)
    mesh = plsc.VectorSubcoreMesh(core_axis_name="c", subcore_axis_name="s")
    nc = 2

    @functools.partial(
        pl.kernel,
        mesh=mesh,
        out_type=jax.ShapeDtypeStruct((rows, d), jnp.float32),
        scratch_types=[
            pltpu.VMEM((n_pass, c_per_pass, chunk), jnp.int32),
            pltpu.VMEM((r_per_pass, d), jnp.float32),
            pltpu.SemaphoreType.DMA,
        ],
        compiler_params=pltpu.CompilerParams(use_tc_tiling_on_sc=False),
    )
    def gather_kernel(table_hbm, idx_hbm, out_hbm, idx_v, rows_v, sem):
        wid = lax.axis_index("s") * nc + lax.axis_index("c")
        pltpu.sync_copy(idx_hbm.at[wid], idx_v)
        for p in range(n_pass):
            copies = []
            for j in range(c_per_pass):
                copies.append(
                    pltpu.async_copy(
                        table_hbm.at[idx_v.at[p, j]],
                        rows_v.at[pl.ds(j * chunk, chunk)],
                        sem,
                    )
                )
            for c in copies:
                c.wait()
            pltpu.sync_copy(
                rows_v,
                out_hbm.at[pl.ds((wid * n_pass + p) * r_per_pass, r_per_pass)],
            )

    return gather_kernel(emb_table, idx4d).reshape(batch, _SLOTS, 128)


_C = 32768


def _repack_body(t_ref, o_ref):
    c = t_ref.shape[1]
    for q in range(c // 512):
        blk = jnp.concatenate(
            [t_ref[:, q * 512 + 128 * k : q * 512 + 128 * (k + 1)] for k in range(4)],
            axis=0,
        )
        o_ref[q * 128 : (q + 1) * 128, :] = blk.T


@jax.jit
def _tc_repack(tT):
    d, vocab = tT.shape
    n_i = -(-vocab // _C)
    return pl.pallas_call(
        _repack_body,
        grid=(n_i,),
        in_specs=[pl.BlockSpec((d, _C), lambda i: (0, i))],
        out_specs=pl.BlockSpec((_C // 4, 128), lambda i: (i, 0)),
        out_shape=jax.ShapeDtypeStruct((n_i * _C // 4, 128), jnp.float32),
    )(tT)


def _linear_body(f_ref, w_ref, b_ref, o_ref):
    acc = b_ref[...]
    for s in range(_SLOTS):
        f = jnp.maximum(f_ref[:, s, :], 0.0)
        acc = acc + jnp.dot(f, w_ref[s], preferred_element_type=jnp.float32)
    o_ref[...] = acc


@jax.jit
def _tc_linear(feats, wr, b):
    batch = feats.shape[0]
    t = wr.shape[2]
    bt = 512
    grid = (batch // bt,)
    return pl.pallas_call(
        _linear_body,
        grid=grid,
        in_specs=[
            pl.BlockSpec((bt, _SLOTS, 128), lambda i: (i, 0, 0)),
            pl.BlockSpec((_SLOTS, 128, t), lambda i: (0, 0, 0)),
            pl.BlockSpec((1, t), lambda i: (0, 0)),
        ],
        out_specs=pl.BlockSpec((bt, t), lambda i: (i, 0)),
        out_shape=jax.ShapeDtypeStruct((batch, t), jnp.float32),
    )(feats, wr, b.reshape(1, t))


def kernel(x, emb_table, W, b):
    batch, inp = x.shape
    _, d = emb_table.shape
    t = W.shape[0]
    slots = _SLOTS * 128 // d
    xp = jnp.concatenate(
        [x, jnp.broadcast_to(x[:, :1], (batch, slots - inp))], axis=1
    )
    fx = ((xp >> 9) << 9) + ((xp & 127) << 2) + ((xp >> 7) & 3)
    rows = batch * slots
    n_pass = 2
    c_per_pass = rows // (_NW * n_pass * _CHUNK)
    idx4d = fx.reshape(_NW, n_pass, c_per_pass, _CHUNK)
    t2 = _tc_repack(jnp.swapaxes(emb_table, 0, 1))
    t_lin = jnp.reshape(t2, (t2.shape[0] * 4, d))
    feats = _sc_gather(t_lin, idx4d)
    wr = jnp.pad(W.T, ((0, (slots - inp) * d), (0, 0))).reshape(_SLOTS, 128, t)
    return _tc_linear(feats, wr, b)


# speedup vs baseline: 21.0239x; 1.0094x over previous
import functools

import jax
import jax.numpy as jnp
from jax import lax
from jax.experimental import pallas as pl
from jax.experimental.pallas import tpu as pltpu
from jax.experimental.pallas import tpu_sc as plsc

_CHUNK = 128
_NW = 32
_SLOTS = 8


@jax.jit
def _sc_gather(emb_table, idx4d):
    _, n_pass, c_per_pass, chunk = idx4d.shape
    rows = _NW * n_pass * c_per_pass * chunk
    _, d = emb_table.shape
    r_per_pass = c_per_pass * chunk
    b_per_pass = r_per_pass * d // (_SLOTS * 128)
    batch = rows * d // (_SLOTS * 128)
    mesh = plsc.VectorSubcoreMesh(core_axis_name="c", subcore_axis_name="s")
    nc = 2

    @functools.partial(
        pl.kernel,
        mesh=mesh,
        out_type=jax.ShapeDtypeStruct((rows, d), jnp.float32),
        scratch_types=[
            pltpu.VMEM((n_pass, c_per_pass, chunk), jnp.int32),
            pltpu.VMEM((r_per_pass, d), jnp.float32),
            pltpu.SemaphoreType.DMA,
        ],
        compiler_params=pltpu.CompilerParams(use_tc_tiling_on_sc=False),
    )
    def gather_kernel(table_hbm, idx_hbm, out_hbm, idx_v, rows_v, sem):
        wid = lax.axis_index("s") * nc + lax.axis_index("c")
        pltpu.sync_copy(idx_hbm.at[wid], idx_v)
        for p in range(n_pass):
            copies = []
            for j in range(c_per_pass):
                copies.append(
                    pltpu.async_copy(
                        table_hbm.at[idx_v.at[p, j]],
                        rows_v.at[pl.ds(j * chunk, chunk)],
                        sem,
                    )
                )
            for c in copies:
                c.wait()
            pltpu.sync_copy(
                rows_v,
                out_hbm.at[pl.ds((wid * n_pass + p) * r_per_pass, r_per_pass)],
            )

    return gather_kernel(emb_table, idx4d).reshape(batch, _SLOTS, 128)


_C = 65536


def _repack_body(t_ref, o_ref):
    c = t_ref.shape[1]
    for q in range(c // 512):
        blk = jnp.concatenate(
            [t_ref[:, q * 512 + 128 * k : q * 512 + 128 * (k + 1)] for k in range(4)],
            axis=0,
        )
        o_ref[q * 128 : (q + 1) * 128, :] = blk.T


@jax.jit
def _tc_repack(tT):
    d, vocab = tT.shape
    n_i = -(-vocab // _C)
    return pl.pallas_call(
        _repack_body,
        grid=(n_i,),
        in_specs=[pl.BlockSpec((d, _C), lambda i: (0, i))],
        out_specs=pl.BlockSpec((_C // 4, 128), lambda i: (i, 0)),
        out_shape=jax.ShapeDtypeStruct((n_i * _C // 4, 128), jnp.float32),
    )(tT)


def _linear_body(f_ref, w_ref, b_ref, o_ref):
    acc = b_ref[...]
    for s in range(_SLOTS):
        f = jnp.maximum(f_ref[:, s, :], 0.0)
        acc = acc + jnp.dot(f, w_ref[s], preferred_element_type=jnp.float32)
    o_ref[...] = acc


@jax.jit
def _tc_linear(feats, wr, b):
    batch = feats.shape[0]
    t = wr.shape[2]
    bt = 512
    grid = (batch // bt,)
    return pl.pallas_call(
        _linear_body,
        grid=grid,
        in_specs=[
            pl.BlockSpec((bt, _SLOTS, 128), lambda i: (i, 0, 0)),
            pl.BlockSpec((_SLOTS, 128, t), lambda i: (0, 0, 0)),
            pl.BlockSpec((1, t), lambda i: (0, 0)),
        ],
        out_specs=pl.BlockSpec((bt, t), lambda i: (i, 0)),
        out_shape=jax.ShapeDtypeStruct((batch, t), jnp.float32),
    )(feats, wr, b.reshape(1, t))


def kernel(x, emb_table, W, b):
    batch, inp = x.shape
    _, d = emb_table.shape
    t = W.shape[0]
    slots = _SLOTS * 128 // d
    xp = jnp.concatenate(
        [x, jnp.broadcast_to(x[:, :1], (batch, slots - inp))], axis=1
    )
    fx = ((xp >> 9) << 9) + ((xp & 127) << 2) + ((xp >> 7) & 3)
    rows = batch * slots
    n_pass = 2
    c_per_pass = rows // (_NW * n_pass * _CHUNK)
    idx4d = fx.reshape(_NW, n_pass, c_per_pass, _CHUNK)
    t2 = _tc_repack(jnp.swapaxes(emb_table, 0, 1))
    t_lin = jnp.reshape(t2, (t2.shape[0] * 4, d))
    feats = _sc_gather(t_lin, idx4d)
    wr = jnp.pad(W.T, ((0, (slots - inp) * d), (0, 0))).reshape(_SLOTS, 128, t)
    return _tc_linear(feats, wr, b)
